# Initial kernel scaffold; baseline (speedup 1.0000x reference)
import jax
import jax.numpy as jnp
from jax.experimental import pallas as pl


def kernel(x, edge_index, W1, b1, W2, b2, W3, b3, Wl, bl):
    raise NotImplementedError("write your pallas kernel here")


# speedup vs baseline: 20.5284x; 20.5284x over previous
import functools

import jax
import jax.numpy as jnp
from jax import lax
from jax.experimental import pallas as pl
from jax.experimental.pallas import tpu as pltpu
from jax.experimental.pallas import tpu_sc as plsc

N = 10000
E = 320000
D = 128
H = 128
OUT = 128

NC = 2
NS = 16
NW = NC * NS

CH = 125
ROWS = E // CH
RPT = ROWS // NW
BLK = 16
NBLK = RPT // BLK
OWN = 632
LAST = N - (NS - 1) * OWN

_mesh = plsc.VectorSubcoreMesh(core_axis_name="c", subcore_axis_name="s")


@functools.partial(
    pl.kernel,
    mesh=_mesh,
    out_type=jax.ShapeDtypeStruct((NC, N, 16), jnp.float32),
    scratch_types=[
        pltpu.VMEM((BLK, CH), jnp.int32),
        pltpu.VMEM((CH, 16), jnp.float32),
        pltpu.VMEM_SHARED((N, 16), jnp.float32),
    ],
)
def _deg_kernel(dst_hbm, zeros_hbm, out_hbm, idx_v, ones_v, acc_sh):
    c = lax.axis_index("c")
    s = lax.axis_index("s")
    t = c * NS + s

    def fill_ones(r, _):
        ones_v[r, :] = jnp.ones((16,), jnp.float32)
        return 0

    lax.fori_loop(0, CH, fill_ones, 0)

    @pl.when(s < NS - 1)
    def _():
        pltpu.sync_copy(zeros_hbm, acc_sh.at[pl.ds(s * OWN, OWN)])

    @pl.when(s == NS - 1)
    def _():
        pltpu.sync_copy(zeros_hbm.at[pl.ds(0, LAST)],
                        acc_sh.at[pl.ds((NS - 1) * OWN, LAST)])

    plsc.subcore_barrier()

    def blk_body(blk, _):
        pltpu.sync_copy(dst_hbm.at[pl.ds(t * RPT + blk * BLK, BLK)], idx_v)

        def ch_body(k, _):
            pltpu.sync_copy(ones_v, acc_sh.at[idx_v.at[k]], add=True)
            return 0

        lax.fori_loop(0, BLK, ch_body, 0)
        return 0

    lax.fori_loop(0, NBLK, blk_body, 0)
    plsc.subcore_barrier()

    @pl.when(s < NS - 1)
    def _():
        pltpu.sync_copy(acc_sh.at[pl.ds(s * OWN, OWN)],
                        out_hbm.at[c, pl.ds(s * OWN, OWN)])

    @pl.when(s == NS - 1)
    def _():
        pltpu.sync_copy(acc_sh.at[pl.ds((NS - 1) * OWN, LAST)],
                        out_hbm.at[c, pl.ds((NS - 1) * OWN, LAST)])


@functools.partial(
    pl.kernel,
    mesh=_mesh,
    out_type=jax.ShapeDtypeStruct((NC, N, H), jnp.float32),
    scratch_types=[
        pltpu.VMEM((BLK, CH), jnp.int32),
        pltpu.VMEM((BLK, CH), jnp.int32),
        pltpu.VMEM((CH, H), jnp.float32),
        pltpu.VMEM((CH, H), jnp.float32),
        pltpu.VMEM_SHARED((N, H), jnp.float32),
        pltpu.SemaphoreType.DMA,
        pltpu.SemaphoreType.DMA,
    ],
)
def _scatter_kernel(g_hbm, src_hbm, dst_hbm, zeros_hbm, out_hbm,
                    src_v, dst_v, rows0, rows1, acc_sh, sem0, sem1):
    c = lax.axis_index("c")
    s = lax.axis_index("s")
    t = c * NS + s

    @pl.when(s < NS - 1)
    def _():
        pltpu.sync_copy(zeros_hbm, acc_sh.at[pl.ds(s * OWN, OWN)])

    @pl.when(s == NS - 1)
    def _():
        pltpu.sync_copy(zeros_hbm.at[pl.ds(0, LAST)],
                        acc_sh.at[pl.ds((NS - 1) * OWN, LAST)])

    plsc.subcore_barrier()

    def blk_body(blk, _):
        pltpu.sync_copy(src_hbm.at[pl.ds(t * RPT + blk * BLK, BLK)], src_v)
        pltpu.sync_copy(dst_hbm.at[pl.ds(t * RPT + blk * BLK, BLK)], dst_v)

        pltpu.async_copy(g_hbm.at[src_v.at[0]], rows0, sem0)

        def ch_body(k, _):
            even = lax.rem(k, 2) == 0

            @pl.when(even)
            def _():
                pltpu.make_async_copy(g_hbm.at[src_v.at[0]], rows0, sem0).wait()

                @pl.when(k + 1 < BLK)
                def _():
                    pltpu.async_copy(g_hbm.at[src_v.at[k + 1]], rows1, sem1)
                pltpu.sync_copy(rows0, acc_sh.at[dst_v.at[k]], add=True)

            @pl.when(jnp.logical_not(even))
            def _():
                pltpu.make_async_copy(g_hbm.at[src_v.at[0]], rows1, sem1).wait()

                @pl.when(k + 1 < BLK)
                def _():
                    pltpu.async_copy(g_hbm.at[src_v.at[k + 1]], rows0, sem0)
                pltpu.sync_copy(rows1, acc_sh.at[dst_v.at[k]], add=True)

            return 0

        lax.fori_loop(0, BLK, ch_body, 0)
        return 0

    lax.fori_loop(0, NBLK, blk_body, 0)
    plsc.subcore_barrier()

    @pl.when(s < NS - 1)
    def _():
        pltpu.sync_copy(acc_sh.at[pl.ds(s * OWN, OWN)],
                        out_hbm.at[c, pl.ds(s * OWN, OWN)])

    @pl.when(s == NS - 1)
    def _():
        pltpu.sync_copy(acc_sh.at[pl.ds((NS - 1) * OWN, LAST)],
                        out_hbm.at[c, pl.ds((NS - 1) * OWN, LAST)])


BN = 2000
GRID = N // BN


def _tc1_body(dega_ref, degb_ref, x_ref, w1_ref, wl0_ref,
              dinv_ref, g1_ref, jk_ref):
    deg = dega_ref[:, 0:1] + degb_ref[:, 0:1] + 1.0
    dinv = lax.rsqrt(deg)
    dinv_ref[...] = dinv
    xb = x_ref[...]
    g1_ref[...] = jnp.dot(xb, w1_ref[...], preferred_element_type=jnp.float32) * dinv
    jk_ref[...] = jnp.dot(xb, wl0_ref[...], preferred_element_type=jnp.float32)


def _tc_mid_body(acca_ref, accb_ref, g_ref, dinv_ref, b_ref, w_ref, wl_ref,
                 jkin_ref, gn_ref, jk_ref):
    dinv = dinv_ref[...]
    h = jnp.maximum(
        dinv * (acca_ref[...] + accb_ref[...] + g_ref[...]) + b_ref[...], 0.0)
    gn_ref[...] = jnp.dot(h, w_ref[...], preferred_element_type=jnp.float32) * dinv
    jk_ref[...] = jkin_ref[...] + jnp.dot(h, wl_ref[...],
                                          preferred_element_type=jnp.float32)


def _tc_fin_body(acca_ref, accb_ref, g_ref, dinv_ref, b_ref, wl_ref, bl_ref,
                 jkin_ref, out_ref):
    h = jnp.maximum(
        dinv_ref[...] * (acca_ref[...] + accb_ref[...] + g_ref[...]) + b_ref[...],
        0.0)
    out_ref[...] = (jkin_ref[...]
                    + jnp.dot(h, wl_ref[...], preferred_element_type=jnp.float32)
                    + bl_ref[...])


def _row_spec(w):
    return pl.BlockSpec((BN, w), lambda i: (i, 0))


def _full_spec(r, c):
    return pl.BlockSpec((r, c), lambda i: (0, 0))


_tc1 = pl.pallas_call(
    _tc1_body,
    grid=(GRID,),
    in_specs=[_row_spec(16), _row_spec(16), _row_spec(D),
              _full_spec(D, H), _full_spec(D, OUT)],
    out_specs=[_row_spec(1), _row_spec(H), _row_spec(OUT)],
    out_shape=[jax.ShapeDtypeStruct((N, 1), jnp.float32),
               jax.ShapeDtypeStruct((N, H), jnp.float32),
               jax.ShapeDtypeStruct((N, OUT), jnp.float32)],
)

_tc_mid = pl.pallas_call(
    _tc_mid_body,
    grid=(GRID,),
    in_specs=[_row_spec(H), _row_spec(H), _row_spec(H), _row_spec(1),
              _full_spec(1, H), _full_spec(H, H), _full_spec(H, OUT),
              _row_spec(OUT)],
    out_specs=[_row_spec(H), _row_spec(OUT)],
    out_shape=[jax.ShapeDtypeStruct((N, H), jnp.float32),
               jax.ShapeDtypeStruct((N, OUT), jnp.float32)],
)

_tc_fin = pl.pallas_call(
    _tc_fin_body,
    grid=(GRID,),
    in_specs=[_row_spec(H), _row_spec(H), _row_spec(H), _row_spec(1),
              _full_spec(1, H), _full_spec(H, OUT), _full_spec(1, OUT),
              _row_spec(OUT)],
    out_specs=_row_spec(OUT),
    out_shape=jax.ShapeDtypeStruct((N, OUT), jnp.float32),
)


def kernel(x, edge_index, W1, b1, W2, b2, W3, b3, Wl, bl):
    src2d = edge_index[0].reshape(ROWS, CH)
    dst2d = edge_index[1].reshape(ROWS, CH)
    zeros16 = jnp.zeros((OWN, 16), jnp.float32)
    zerosH = jnp.zeros((OWN, H), jnp.float32)

    degp = _deg_kernel(dst2d, zeros16)
    dega, degb = degp[0], degp[1]

    dinv, g1, jk = _tc1(dega, degb, x, W1, Wl[0:D])

    acc = _scatter_kernel(g1, src2d, dst2d, zerosH)
    g2, jk = _tc_mid(acc[0], acc[1], g1, dinv, b1.reshape(1, H), W2,
                     Wl[D:D + H], jk)

    acc = _scatter_kernel(g2, src2d, dst2d, zerosH)
    g3, jk = _tc_mid(acc[0], acc[1], g2, dinv, b2.reshape(1, H), W3,
                     Wl[D + H:D + 2 * H], jk)

    acc = _scatter_kernel(g3, src2d, dst2d, zerosH)
    out = _tc_fin(acc[0], acc[1], g3, dinv, b3.reshape(1, H),
                  Wl[D + 2 * H:], bl.reshape(1, OUT), jk)
    return out


# speedup vs baseline: 24.3584x; 1.1866x over previous
import functools

import jax
import jax.numpy as jnp
from jax import lax
from jax.experimental import pallas as pl
from jax.experimental.pallas import tpu as pltpu
from jax.experimental.pallas import tpu_sc as plsc

N = 10000
E = 320000
D = 128
H = 128
OUT = 128

NC = 2
NS = 16
NW = NC * NS

CH = 125
ROWS = E // CH
RPT = ROWS // NW
BLK = 16
NBLK = RPT // BLK
OWN = 632
LAST = N - (NS - 1) * OWN

_mesh = plsc.VectorSubcoreMesh(core_axis_name="c", subcore_axis_name="s")


@functools.partial(
    pl.kernel,
    mesh=_mesh,
    out_type=jax.ShapeDtypeStruct((NC, N, 16), jnp.float32),
    scratch_types=[
        pltpu.VMEM((BLK, CH), jnp.int32),
        pltpu.VMEM((CH, 16), jnp.float32),
        pltpu.VMEM_SHARED((N, 16), jnp.float32),
    ],
)
def _deg_kernel(dst_hbm, zeros_hbm, out_hbm, idx_v, ones_v, acc_sh):
    c = lax.axis_index("c")
    s = lax.axis_index("s")
    t = c * NS + s

    def fill_ones(r, _):
        ones_v[r, :] = jnp.ones((16,), jnp.float32)
        return 0

    lax.fori_loop(0, CH, fill_ones, 0)

    @pl.when(s < NS - 1)
    def _():
        pltpu.sync_copy(zeros_hbm, acc_sh.at[pl.ds(s * OWN, OWN)])

    @pl.when(s == NS - 1)
    def _():
        pltpu.sync_copy(zeros_hbm.at[pl.ds(0, LAST)],
                        acc_sh.at[pl.ds((NS - 1) * OWN, LAST)])

    plsc.subcore_barrier()

    def blk_body(blk, _):
        pltpu.sync_copy(dst_hbm.at[pl.ds(t * RPT + blk * BLK, BLK)], idx_v)

        def ch_body(k, _):
            pltpu.sync_copy(ones_v, acc_sh.at[idx_v.at[k]], add=True)
            return 0

        lax.fori_loop(0, BLK, ch_body, 0)
        return 0

    lax.fori_loop(0, NBLK, blk_body, 0)
    plsc.subcore_barrier()

    @pl.when(s < NS - 1)
    def _():
        pltpu.sync_copy(acc_sh.at[pl.ds(s * OWN, OWN)],
                        out_hbm.at[c, pl.ds(s * OWN, OWN)])

    @pl.when(s == NS - 1)
    def _():
        pltpu.sync_copy(acc_sh.at[pl.ds((NS - 1) * OWN, LAST)],
                        out_hbm.at[c, pl.ds((NS - 1) * OWN, LAST)])


@functools.partial(
    pl.kernel,
    mesh=_mesh,
    out_type=jax.ShapeDtypeStruct((NC, N, H), jnp.float32),
    scratch_types=[
        pltpu.VMEM((2 * BLK, CH), jnp.int32),
        pltpu.VMEM((2 * BLK, CH), jnp.int32),
        pltpu.VMEM((CH, H), jnp.float32),
        pltpu.VMEM((CH, H), jnp.float32),
        pltpu.SemaphoreType.DMA,
        pltpu.SemaphoreType.DMA,
        pltpu.SemaphoreType.DMA,
        pltpu.SemaphoreType.DMA,
        pltpu.SemaphoreType.DMA,
        pltpu.VMEM_SHARED((N, H), jnp.float32),
    ],
)
def _scatter_kernel(g_hbm, src_hbm, dst_hbm, zeros_hbm, out_hbm,
                    src_v, dst_v, buf0, buf1,
                    sg0, sg1, ss0, ss1, si, acc_sh):
    c = lax.axis_index("c")
    s = lax.axis_index("s")
    t = c * NS + s
    bufs = (buf0, buf1)
    sgs = (sg0, sg1)
    sss = (ss0, ss1)

    @pl.when(s < NS - 1)
    def _():
        pltpu.sync_copy(zeros_hbm, acc_sh.at[pl.ds(s * OWN, OWN)])

    @pl.when(s == NS - 1)
    def _():
        pltpu.sync_copy(zeros_hbm.at[pl.ds(0, LAST)],
                        acc_sh.at[pl.ds((NS - 1) * OWN, LAST)])

    plsc.subcore_barrier()

    def _load_idx_async(b):
        off = pl.multiple_of(lax.rem(b, 2) * BLK, BLK)
        base = t * RPT + b * BLK
        pltpu.async_copy(src_hbm.at[pl.ds(base, BLK)],
                         src_v.at[pl.ds(off, BLK)], si)
        pltpu.async_copy(dst_hbm.at[pl.ds(base, BLK)],
                         dst_v.at[pl.ds(off, BLK)], si)

    def _wait_idx():
        pltpu.make_async_copy(src_hbm.at[pl.ds(0, BLK)],
                              src_v.at[pl.ds(0, BLK)], si).wait()
        pltpu.make_async_copy(dst_hbm.at[pl.ds(0, BLK)],
                              dst_v.at[pl.ds(0, BLK)], si).wait()

    def _wait_gather(p):
        pltpu.make_async_copy(g_hbm.at[src_v.at[0]], bufs[p], sgs[p]).wait()

    def _wait_scatter(p):
        pltpu.make_async_copy(bufs[p], acc_sh.at[dst_v.at[0]],
                              sss[p]).wait()

    _load_idx_async(0)
    _wait_idx()
    _load_idx_async(1)

    def blk_body(b, _):
        @pl.when(b >= 1)
        def _():
            _wait_idx()

        for j in range(BLK):
            p = j % 2
            row = lax.rem(b, 2) * BLK + j
            pltpu.async_copy(g_hbm.at[src_v.at[row]], bufs[p], sgs[p])

            def _scatter_prev():
                k = b * BLK + j
                prow = lax.rem(k - 1, 2 * BLK)
                _wait_gather(1 - p)
                pltpu.sync_copy(bufs[1 - p], acc_sh.at[dst_v.at[prow]],
                                add=True)

            if j >= 1:
                _scatter_prev()
            else:
                @pl.when(b > 0)
                def _():
                    _scatter_prev()

            if j == 1:
                @pl.when(jnp.logical_and(b >= 1, b < NBLK - 1))
                def _():
                    _load_idx_async(b + 1)
        return 0

    lax.fori_loop(0, NBLK, blk_body, 0)

    pl_last = (RPT - 1) % 2
    _wait_gather(pl_last)
    pltpu.sync_copy(bufs[pl_last],
                    acc_sh.at[dst_v.at[lax.rem(RPT - 1, 2 * BLK)]],
                    add=True)

    plsc.subcore_barrier()

    @pl.when(s < NS - 1)
    def _():
        pltpu.sync_copy(acc_sh.at[pl.ds(s * OWN, OWN)],
                        out_hbm.at[c, pl.ds(s * OWN, OWN)])

    @pl.when(s == NS - 1)
    def _():
        pltpu.sync_copy(acc_sh.at[pl.ds((NS - 1) * OWN, LAST)],
                        out_hbm.at[c, pl.ds((NS - 1) * OWN, LAST)])


BN = 2000
GRID = N // BN


def _tc1_body(dega_ref, degb_ref, x_ref, w1_ref, wl0_ref,
              dinv_ref, g1_ref, jk_ref):
    deg = dega_ref[:, 0:1] + degb_ref[:, 0:1] + 1.0
    dinv = lax.rsqrt(deg)
    dinv_ref[...] = dinv
    xb = x_ref[...]
    g1_ref[...] = jnp.dot(xb, w1_ref[...], preferred_element_type=jnp.float32) * dinv
    jk_ref[...] = jnp.dot(xb, wl0_ref[...], preferred_element_type=jnp.float32)


def _tc_mid_body(acca_ref, accb_ref, g_ref, dinv_ref, b_ref, w_ref, wl_ref,
                 jkin_ref, gn_ref, jk_ref):
    dinv = dinv_ref[...]
    h = jnp.maximum(
        dinv * (acca_ref[...] + accb_ref[...] + g_ref[...]) + b_ref[...], 0.0)
    gn_ref[...] = jnp.dot(h, w_ref[...], preferred_element_type=jnp.float32) * dinv
    jk_ref[...] = jkin_ref[...] + jnp.dot(h, wl_ref[...],
                                          preferred_element_type=jnp.float32)


def _tc_fin_body(acca_ref, accb_ref, g_ref, dinv_ref, b_ref, wl_ref, bl_ref,
                 jkin_ref, out_ref):
    h = jnp.maximum(
        dinv_ref[...] * (acca_ref[...] + accb_ref[...] + g_ref[...]) + b_ref[...],
        0.0)
    out_ref[...] = (jkin_ref[...]
                    + jnp.dot(h, wl_ref[...], preferred_element_type=jnp.float32)
                    + bl_ref[...])


def _row_spec(w):
    return pl.BlockSpec((BN, w), lambda i: (i, 0))


def _full_spec(r, c):
    return pl.BlockSpec((r, c), lambda i: (0, 0))


_tc1 = pl.pallas_call(
    _tc1_body,
    grid=(GRID,),
    in_specs=[_row_spec(16), _row_spec(16), _row_spec(D),
              _full_spec(D, H), _full_spec(D, OUT)],
    out_specs=[_row_spec(1), _row_spec(H), _row_spec(OUT)],
    out_shape=[jax.ShapeDtypeStruct((N, 1), jnp.float32),
               jax.ShapeDtypeStruct((N, H), jnp.float32),
               jax.ShapeDtypeStruct((N, OUT), jnp.float32)],
)

_tc_mid = pl.pallas_call(
    _tc_mid_body,
    grid=(GRID,),
    in_specs=[_row_spec(H), _row_spec(H), _row_spec(H), _row_spec(1),
              _full_spec(1, H), _full_spec(H, H), _full_spec(H, OUT),
              _row_spec(OUT)],
    out_specs=[_row_spec(H), _row_spec(OUT)],
    out_shape=[jax.ShapeDtypeStruct((N, H), jnp.float32),
               jax.ShapeDtypeStruct((N, OUT), jnp.float32)],
)

_tc_fin = pl.pallas_call(
    _tc_fin_body,
    grid=(GRID,),
    in_specs=[_row_spec(H), _row_spec(H), _row_spec(H), _row_spec(1),
              _full_spec(1, H), _full_spec(H, OUT), _full_spec(1, OUT),
              _row_spec(OUT)],
    out_specs=_row_spec(OUT),
    out_shape=jax.ShapeDtypeStruct((N, OUT), jnp.float32),
)


def kernel(x, edge_index, W1, b1, W2, b2, W3, b3, Wl, bl):
    src2d = edge_index[0].reshape(ROWS, CH)
    dst2d = edge_index[1].reshape(ROWS, CH)
    zeros16 = jnp.zeros((OWN, 16), jnp.float32)
    zerosH = jnp.zeros((OWN, H), jnp.float32)

    degp = _deg_kernel(dst2d, zeros16)
    dega, degb = degp[0], degp[1]

    dinv, g1, jk = _tc1(dega, degb, x, W1, Wl[0:D])

    acc = _scatter_kernel(g1, src2d, dst2d, zerosH)
    g2, jk = _tc_mid(acc[0], acc[1], g1, dinv, b1.reshape(1, H), W2,
                     Wl[D:D + H], jk)

    acc = _scatter_kernel(g2, src2d, dst2d, zerosH)
    g3, jk = _tc_mid(acc[0], acc[1], g2, dinv, b2.reshape(1, H), W3,
                     Wl[D + H:D + 2 * H], jk)

    acc = _scatter_kernel(g3, src2d, dst2d, zerosH)
    out = _tc_fin(acc[0], acc[1], g3, dinv, b3.reshape(1, H),
                  Wl[D + 2 * H:], bl.reshape(1, OUT), jk)
    return out
